# baseline (speedup 1.0000x reference)
import functools

import jax
import jax.numpy as jnp
from jax import lax
from jax.experimental import pallas as pl
from jax.experimental.pallas import tpu as pltpu
from jax.experimental.pallas import tpu_sc as plsc

N = 10000
E = 320000
D = 128
NC = 2
NS = 16
B = 128
NCH = 80
NB = 4
E_PAD = NC * NS * NCH * B
NPAD = 10112
RPT = NPAD // NS

_mesh = plsc.VectorSubcoreMesh(core_axis_name="c", subcore_axis_name="s")
_sc_params = pltpu.CompilerParams(needs_layout_passes=False)


@functools.partial(
    pl.kernel,
    out_type=jax.ShapeDtypeStruct((NC, NPAD, D), jnp.float32),
    mesh=_mesh,
    scratch_types=[
        pltpu.VMEM((B,), jnp.int32),
        pltpu.VMEM((B, D), jnp.float32),
        pltpu.VMEM_SHARED((NPAD, D), jnp.float32),
    ],
    compiler_params=_sc_params,
)
def _deg_kernel(dst_hbm, ones_hbm, z128_hbm, deg_out, didx_v, ones_v, deg_sh):
    c = lax.axis_index("c")
    s = lax.axis_index("s")
    wid = c * NS + s
    r0 = s * RPT
    pltpu.sync_copy(ones_hbm, ones_v)
    pltpu.sync_copy(z128_hbm.at[pl.ds(r0, RPT)], deg_sh.at[pl.ds(r0, RPT)])
    plsc.subcore_barrier()

    def step(k, carry):
        pltpu.sync_copy(dst_hbm.at[wid, k], didx_v)
        pltpu.sync_copy(ones_v, deg_sh.at[didx_v], add=True)
        return carry

    lax.fori_loop(0, NCH, step, 0)
    plsc.subcore_barrier()
    pltpu.sync_copy(deg_sh.at[pl.ds(r0, RPT)], deg_out.at[c, pl.ds(r0, RPT)])


@functools.partial(
    pl.kernel,
    out_type=jax.ShapeDtypeStruct((NC, NPAD, D), jnp.float32),
    mesh=_mesh,
    scratch_types=[
        pltpu.VMEM((NB, B), jnp.int32),
        pltpu.VMEM((NB, B), jnp.int32),
        pltpu.VMEM((B, D), jnp.float32),
        pltpu.VMEM_SHARED((NPAD, D), jnp.float32),
        pltpu.SemaphoreType.DMA,
    ],
    compiler_params=_sc_params,
)
def _edge_kernel(gs_hbm, src_hbm, dst_hbm, z128_hbm, acc_out,
                 sidx_v, didx_v, rows_v, acc_sh, gsem):
    c = lax.axis_index("c")
    s = lax.axis_index("s")
    wid = c * NS + s
    r0 = s * RPT
    pltpu.sync_copy(z128_hbm.at[pl.ds(r0, RPT)], acc_sh.at[pl.ds(r0, RPT)])
    plsc.subcore_barrier()

    def step(k, carry):
        pltpu.sync_copy(src_hbm.at[wid, pl.ds(k * NB, NB)], sidx_v)
        pltpu.sync_copy(dst_hbm.at[wid, pl.ds(k * NB, NB)], didx_v)
        for h in range(NB):
            pltpu.async_copy(gs_hbm.at[sidx_v.at[h]], rows_v, gsem).wait()
            pltpu.sync_copy(rows_v, acc_sh.at[didx_v.at[h]], add=True)
        return carry

    lax.fori_loop(0, NCH // NB, step, 0)
    plsc.subcore_barrier()
    pltpu.sync_copy(acc_sh.at[pl.ds(r0, RPT)], acc_out.at[c, pl.ds(r0, RPT)])


@functools.partial(
    pl.kernel,
    out_type=jax.ShapeDtypeStruct((NC, NS, 16), jnp.float32),
    mesh=_mesh,
    scratch_types=[
        pltpu.VMEM((NCH * B,), jnp.int32),
        pltpu.VMEM((NCH * B,), jnp.int32),
        pltpu.VMEM((NPAD,), jnp.float32),
        pltpu.VMEM((NPAD,), jnp.float32),
        pltpu.VMEM((16,), jnp.float32),
    ],
    compiler_params=_sc_params,
)
def _csum_kernel(u_hbm, dinv_hbm, src_hbm, dst_hbm, part_out,
                 src_v, dst_v, u_v, dinv_v, acc_v):
    c = lax.axis_index("c")
    s = lax.axis_index("s")
    wid = c * NS + s
    pltpu.sync_copy(src_hbm.at[wid], src_v)
    pltpu.sync_copy(dst_hbm.at[wid], dst_v)
    pltpu.sync_copy(u_hbm, u_v)
    pltpu.sync_copy(dinv_hbm, dinv_v)

    def step(j, acc):
        uj = plsc.load_gather(u_v, [src_v[pl.ds(j * 16, 16)]])
        dj = plsc.load_gather(dinv_v, [dst_v[pl.ds(j * 16, 16)]])
        return acc + uj * dj

    acc = lax.fori_loop(0, NCH * B // 16, step, jnp.zeros((16,), jnp.float32))
    acc_v[...] = acc
    pltpu.sync_copy(acc_v, part_out.at[c, s])


def _scale_body(x_ref, w1_ref, degp_ref, gs_ref, d16_ref):
    deg = degp_ref[0, :, 0:16] + degp_ref[1, :, 0:16] + 1.0
    dinv = lax.rsqrt(deg)
    g = jnp.dot(x_ref[...], w1_ref[...],
                preferred_element_type=jnp.float32,
                precision=lax.Precision.HIGHEST)
    gs_ref[...] = g * dinv[:, 0:1]
    d16_ref[...] = dinv


_scale_call = pl.pallas_call(
    _scale_body,
    out_shape=(
        jax.ShapeDtypeStruct((NPAD, D), jnp.float32),
        jax.ShapeDtypeStruct((NPAD, 16), jnp.float32),
    ),
)


def _mid_body(accp_ref, gs_ref, d16_ref, b1_ref, w2_ref, wp_ref,
              u_ref, self_ref):
    acc = accp_ref[0] + accp_ref[1] + gs_ref[...]
    dinv = d16_ref[:, 0:1]
    h1 = jnp.maximum(acc * dinv + b1_ref[...], 0.0)
    w = jnp.dot(w2_ref[...], wp_ref[...],
                preferred_element_type=jnp.float32,
                precision=lax.Precision.HIGHEST)
    t = jnp.dot(h1, w, preferred_element_type=jnp.float32,
                precision=lax.Precision.HIGHEST)
    row = lax.broadcasted_iota(jnp.int32, (NPAD, 1), 0)
    u = jnp.where(row < N, t * dinv, 0.0)
    u_ref[...] = u
    self_ref[...] = jnp.sum(u * dinv, keepdims=True)


_mid_call = pl.pallas_call(
    _mid_body,
    out_shape=(
        jax.ShapeDtypeStruct((NPAD, 1), jnp.float32),
        jax.ShapeDtypeStruct((1, 1), jnp.float32),
    ),
)


def _combine_body(part_ref, self_ref, b2_ref, wp_ref, bp_ref, out_ref):
    edge_sum = jnp.sum(part_ref[...])
    const = jnp.dot(b2_ref[...], wp_ref[...],
                    preferred_element_type=jnp.float32,
                    precision=lax.Precision.HIGHEST)
    out_ref[...] = (edge_sum + self_ref[...]) / float(N) + const + bp_ref[...]


_combine_call = pl.pallas_call(
    _combine_body,
    out_shape=jax.ShapeDtypeStruct((1, 1), jnp.float32),
)


@jax.jit
def kernel(x, edge_index, W1, b1, W2, b2, Wp, bp):
    src = edge_index[0].astype(jnp.int32)
    dst = edge_index[1].astype(jnp.int32)
    pad = jnp.full((E_PAD - E,), N, dtype=jnp.int32)
    src_flat = jnp.concatenate([src, pad])
    dst_flat = jnp.concatenate([dst, pad])
    src_ch = src_flat.reshape(NC * NS, NCH, B)
    dst_ch = dst_flat.reshape(NC * NS, NCH, B)
    src_16 = src_flat.reshape(NC * NS, NCH * B)
    dst_16 = dst_flat.reshape(NC * NS, NCH * B)

    x_pad = jnp.pad(x, ((0, NPAD - N), (0, 0)))
    ones = jnp.ones((B, D), jnp.float32)
    z128 = jnp.zeros((NPAD, D), jnp.float32)

    deg_p = _deg_kernel(dst_ch, ones, z128)
    gs, d16 = _scale_call(x_pad, W1, deg_p)
    acc_p = _edge_kernel(gs, src_ch, dst_ch, z128)
    u, selfterm = _mid_call(acc_p, gs, d16, b1.reshape(1, D), W2, Wp)
    part = _csum_kernel(u.reshape(NPAD), d16[:, 0], src_16, dst_16)
    out = _combine_call(part, selfterm, b2.reshape(1, D), Wp,
                        bp.reshape(1, 1))
    return out.reshape(1)


# speedup vs baseline: 1.2984x; 1.2984x over previous
import functools

import jax
import jax.numpy as jnp
from jax import lax
from jax.experimental import pallas as pl
from jax.experimental.pallas import tpu as pltpu
from jax.experimental.pallas import tpu_sc as plsc

N = 10000
E = 320000
D = 128
NC = 2
NS = 16
B = 128
NCH = 79
E_PAD = NC * NS * NCH * B
NPAD = 10112
RPT = NPAD // NS

_mesh = plsc.VectorSubcoreMesh(core_axis_name="c", subcore_axis_name="s")
_sc_params = pltpu.CompilerParams(needs_layout_passes=False)


@functools.partial(
    pl.kernel,
    out_type=jax.ShapeDtypeStruct((NC, NPAD, D), jnp.float32),
    mesh=_mesh,
    scratch_types=[
        pltpu.VMEM((B,), jnp.int32),
        pltpu.VMEM((B, D), jnp.float32),
        pltpu.VMEM_SHARED((NPAD, D), jnp.float32),
    ],
    compiler_params=_sc_params,
)
def _deg_kernel(dst_hbm, ones_hbm, z128_hbm, deg_out, didx_v, ones_v, deg_sh):
    c = lax.axis_index("c")
    s = lax.axis_index("s")
    wid = c * NS + s
    r0 = s * RPT
    pltpu.sync_copy(ones_hbm, ones_v)
    pltpu.sync_copy(z128_hbm.at[pl.ds(r0, RPT)], deg_sh.at[pl.ds(r0, RPT)])
    plsc.subcore_barrier()

    def step(k, carry):
        pltpu.sync_copy(dst_hbm.at[wid, k], didx_v)
        pltpu.sync_copy(ones_v, deg_sh.at[didx_v], add=True)
        return carry

    lax.fori_loop(0, NCH, step, 0)
    plsc.subcore_barrier()
    pltpu.sync_copy(deg_sh.at[pl.ds(r0, RPT)], deg_out.at[c, pl.ds(r0, RPT)])


@functools.partial(
    pl.kernel,
    out_type=jax.ShapeDtypeStruct((NC, NPAD, D), jnp.float32),
    mesh=_mesh,
    scratch_types=[
        pltpu.VMEM((B,), jnp.int32),
        pltpu.VMEM((B,), jnp.int32),
        pltpu.VMEM((B, D), jnp.float32),
        pltpu.VMEM_SHARED((NPAD, D), jnp.float32),
        pltpu.SemaphoreType.DMA,
    ],
    compiler_params=_sc_params,
)
def _edge_kernel(gs_hbm, src_hbm, dst_hbm, z128_hbm, acc_out,
                 sidx_v, didx_v, rows_v, acc_sh, gsem):
    c = lax.axis_index("c")
    s = lax.axis_index("s")
    wid = c * NS + s
    r0 = s * RPT
    pltpu.sync_copy(z128_hbm.at[pl.ds(r0, RPT)], acc_sh.at[pl.ds(r0, RPT)])
    plsc.subcore_barrier()

    def step(k, carry):
        pltpu.sync_copy(src_hbm.at[wid, k], sidx_v)
        pltpu.sync_copy(dst_hbm.at[wid, k], didx_v)
        pltpu.async_copy(gs_hbm.at[sidx_v], rows_v, gsem).wait()
        pltpu.sync_copy(rows_v, acc_sh.at[didx_v], add=True)
        return carry

    lax.fori_loop(0, NCH, step, 0)
    plsc.subcore_barrier()
    pltpu.sync_copy(acc_sh.at[pl.ds(r0, RPT)], acc_out.at[c, pl.ds(r0, RPT)])


@functools.partial(
    pl.kernel,
    out_type=jax.ShapeDtypeStruct((NC, NS, 16), jnp.float32),
    mesh=_mesh,
    scratch_types=[
        pltpu.VMEM((NCH * B,), jnp.int32),
        pltpu.VMEM((NCH * B,), jnp.int32),
        pltpu.VMEM((NPAD,), jnp.float32),
        pltpu.VMEM((NPAD,), jnp.float32),
        pltpu.VMEM((16,), jnp.float32),
    ],
    compiler_params=_sc_params,
)
def _csum_kernel(u_hbm, dinv_hbm, src_hbm, dst_hbm, part_out,
                 src_v, dst_v, u_v, dinv_v, acc_v):
    c = lax.axis_index("c")
    s = lax.axis_index("s")
    wid = c * NS + s
    pltpu.sync_copy(src_hbm.at[wid], src_v)
    pltpu.sync_copy(dst_hbm.at[wid], dst_v)
    pltpu.sync_copy(u_hbm, u_v)
    pltpu.sync_copy(dinv_hbm, dinv_v)

    def step(j, acc):
        uj = plsc.load_gather(u_v, [src_v[pl.ds(j * 16, 16)]])
        dj = plsc.load_gather(dinv_v, [dst_v[pl.ds(j * 16, 16)]])
        return acc + uj * dj

    acc = lax.fori_loop(0, NCH * B // 16, step, jnp.zeros((16,), jnp.float32))
    acc_v[...] = acc
    pltpu.sync_copy(acc_v, part_out.at[c, s])


def _scale_body(x_ref, w1_ref, degp_ref, gs_ref, d16_ref):
    deg = degp_ref[0, :, 0:16] + degp_ref[1, :, 0:16] + 1.0
    dinv = lax.rsqrt(deg)
    g = jnp.dot(x_ref[...], w1_ref[...],
                preferred_element_type=jnp.float32,
                precision=lax.Precision.HIGHEST)
    gs_ref[...] = g * dinv[:, 0:1]
    d16_ref[...] = dinv


_scale_call = pl.pallas_call(
    _scale_body,
    out_shape=(
        jax.ShapeDtypeStruct((NPAD, D), jnp.float32),
        jax.ShapeDtypeStruct((NPAD, 16), jnp.float32),
    ),
)


def _mid_body(accp_ref, gs_ref, d16_ref, b1_ref, w2_ref, wp_ref,
              u_ref, self_ref):
    acc = accp_ref[0] + accp_ref[1] + gs_ref[...]
    dinv = d16_ref[:, 0:1]
    h1 = jnp.maximum(acc * dinv + b1_ref[...], 0.0)
    w = jnp.dot(w2_ref[...], wp_ref[...],
                preferred_element_type=jnp.float32,
                precision=lax.Precision.HIGHEST)
    t = jnp.dot(h1, w, preferred_element_type=jnp.float32,
                precision=lax.Precision.HIGHEST)
    row = lax.broadcasted_iota(jnp.int32, (NPAD, 1), 0)
    u = jnp.where(row < N, t * dinv, 0.0)
    u_ref[...] = u
    self_ref[...] = jnp.sum(u * dinv, keepdims=True)


_mid_call = pl.pallas_call(
    _mid_body,
    out_shape=(
        jax.ShapeDtypeStruct((NPAD, 1), jnp.float32),
        jax.ShapeDtypeStruct((1, 1), jnp.float32),
    ),
)


def _combine_body(part_ref, self_ref, b2_ref, wp_ref, bp_ref, out_ref):
    edge_sum = jnp.sum(part_ref[...])
    const = jnp.dot(b2_ref[...], wp_ref[...],
                    preferred_element_type=jnp.float32,
                    precision=lax.Precision.HIGHEST)
    out_ref[...] = (edge_sum + self_ref[...]) / float(N) + const + bp_ref[...]


_combine_call = pl.pallas_call(
    _combine_body,
    out_shape=jax.ShapeDtypeStruct((1, 1), jnp.float32),
)


@jax.jit
def kernel(x, edge_index, W1, b1, W2, b2, Wp, bp):
    src = edge_index[0].astype(jnp.int32)
    dst = edge_index[1].astype(jnp.int32)
    pad = jnp.full((E_PAD - E,), N, dtype=jnp.int32)
    src_flat = jnp.concatenate([src, pad])
    dst_flat = jnp.concatenate([dst, pad])
    src_ch = src_flat.reshape(NC * NS, NCH, B)
    dst_ch = dst_flat.reshape(NC * NS, NCH, B)
    src_16 = src_flat.reshape(NC * NS, NCH * B)
    dst_16 = dst_flat.reshape(NC * NS, NCH * B)

    x_pad = jnp.pad(x, ((0, NPAD - N), (0, 0)))
    ones = jnp.ones((B, D), jnp.float32)
    z128 = jnp.zeros((NPAD, D), jnp.float32)

    deg_p = _deg_kernel(dst_ch, ones, z128)
    gs, d16 = _scale_call(x_pad, W1, deg_p)
    acc_p = _edge_kernel(gs, src_ch, dst_ch, z128)
    u, selfterm = _mid_call(acc_p, gs, d16, b1.reshape(1, D), W2, Wp)
    part = _csum_kernel(u.reshape(NPAD), d16[:, 0], src_16, dst_16)
    out = _combine_call(part, selfterm, b2.reshape(1, D), Wp,
                        bp.reshape(1, 1))
    return out.reshape(1)


# speedup vs baseline: 1.3791x; 1.0622x over previous
import functools

import jax
import jax.numpy as jnp
from jax import lax
from jax.experimental import pallas as pl
from jax.experimental.pallas import tpu as pltpu
from jax.experimental.pallas import tpu_sc as plsc

N = 10000
E = 320000
D = 128
NC = 2
NS = 16
B = 128
NCH = 79
E_PAD = NC * NS * NCH * B
NPAD = 10240
RPT = NPAD // NS

_mesh = plsc.VectorSubcoreMesh(core_axis_name="c", subcore_axis_name="s")
_sc_params = pltpu.CompilerParams(needs_layout_passes=False)


@functools.partial(
    pl.kernel,
    out_type=jax.ShapeDtypeStruct((NC, NPAD), jnp.float32),
    mesh=_mesh,
    scratch_types=[
        pltpu.VMEM((NCH * B,), jnp.int32),
        pltpu.VMEM((NPAD,), jnp.float32),
        pltpu.VMEM((RPT,), jnp.float32),
        pltpu.VMEM((RPT,), jnp.float32),
        pltpu.VMEM_SHARED((NS, NPAD), jnp.float32),
    ],
    compiler_params=_sc_params,
)
def _deg_kernel(dst_hbm, zflat_hbm, deg_out,
                dst_v, hist_v, tmp_v, sum_v, hist_sh):
    c = lax.axis_index("c")
    s = lax.axis_index("s")
    wid = c * NS + s
    r0 = s * RPT
    pltpu.sync_copy(dst_hbm.at[wid], dst_v)
    pltpu.sync_copy(zflat_hbm, hist_v)

    def step(j, carry):
        x = dst_v[pl.ds(j * 16, 16)]
        cnt, last = plsc.scan_count(x)
        old = plsc.load_gather(hist_v, [x])
        plsc.store_scatter(hist_v, [x], old + cnt.astype(jnp.float32),
                           mask=last)
        return carry

    lax.fori_loop(0, NCH * B // 16, step, 0)
    pltpu.sync_copy(hist_v, hist_sh.at[s])
    plsc.subcore_barrier()

    pltpu.sync_copy(zflat_hbm.at[pl.ds(0, RPT)], sum_v)

    def merge(t, carry):
        pltpu.sync_copy(hist_sh.at[t, pl.ds(r0, RPT)], tmp_v)

        def add(j, carry2):
            sl = pl.ds(j * 16, 16)
            sum_v[sl] = sum_v[sl] + tmp_v[sl]
            return carry2

        lax.fori_loop(0, RPT // 16, add, 0)
        return carry

    lax.fori_loop(0, NS, merge, 0)
    pltpu.sync_copy(sum_v, deg_out.at[c, pl.ds(r0, RPT)])


@functools.partial(
    pl.kernel,
    out_type=jax.ShapeDtypeStruct((NC, NPAD, D), jnp.float32),
    mesh=_mesh,
    scratch_types=[
        pltpu.VMEM((B,), jnp.int32),
        pltpu.VMEM((B,), jnp.int32),
        pltpu.VMEM((B, D), jnp.float32),
        pltpu.VMEM_SHARED((NPAD, D), jnp.float32),
        pltpu.SemaphoreType.DMA,
    ],
    compiler_params=_sc_params,
)
def _edge_kernel(gs_hbm, src_hbm, dst_hbm, z128_hbm, acc_out,
                 sidx_v, didx_v, rows_v, acc_sh, gsem):
    c = lax.axis_index("c")
    s = lax.axis_index("s")
    wid = c * NS + s
    r0 = s * RPT
    pltpu.sync_copy(z128_hbm.at[pl.ds(r0, RPT)], acc_sh.at[pl.ds(r0, RPT)])
    plsc.subcore_barrier()

    def step(k, carry):
        pltpu.sync_copy(src_hbm.at[wid, k], sidx_v)
        pltpu.sync_copy(dst_hbm.at[wid, k], didx_v)
        pltpu.async_copy(gs_hbm.at[sidx_v], rows_v, gsem).wait()
        pltpu.sync_copy(rows_v, acc_sh.at[didx_v], add=True)
        return carry

    lax.fori_loop(0, NCH, step, 0)
    plsc.subcore_barrier()
    pltpu.sync_copy(acc_sh.at[pl.ds(r0, RPT)], acc_out.at[c, pl.ds(r0, RPT)])


@functools.partial(
    pl.kernel,
    out_type=jax.ShapeDtypeStruct((NC, NS, 16), jnp.float32),
    mesh=_mesh,
    scratch_types=[
        pltpu.VMEM((NCH * B,), jnp.int32),
        pltpu.VMEM((NCH * B,), jnp.int32),
        pltpu.VMEM((NPAD,), jnp.float32),
        pltpu.VMEM((NPAD,), jnp.float32),
        pltpu.VMEM((16,), jnp.float32),
    ],
    compiler_params=_sc_params,
)
def _csum_kernel(u_hbm, dinv_hbm, src_hbm, dst_hbm, part_out,
                 src_v, dst_v, u_v, dinv_v, acc_v):
    c = lax.axis_index("c")
    s = lax.axis_index("s")
    wid = c * NS + s
    pltpu.sync_copy(src_hbm.at[wid], src_v)
    pltpu.sync_copy(dst_hbm.at[wid], dst_v)
    pltpu.sync_copy(u_hbm, u_v)
    pltpu.sync_copy(dinv_hbm, dinv_v)

    def step(j, acc):
        uj = plsc.load_gather(u_v, [src_v[pl.ds(j * 16, 16)]])
        dj = plsc.load_gather(dinv_v, [dst_v[pl.ds(j * 16, 16)]])
        return acc + uj * dj

    acc = lax.fori_loop(0, NCH * B // 16, step, jnp.zeros((16,), jnp.float32))
    acc_v[...] = acc
    pltpu.sync_copy(acc_v, part_out.at[c, s])


def _scale_body(x_ref, w1_ref, deg16_ref, gs_ref, d16_ref):
    deg = deg16_ref[...] + 1.0
    dinv = lax.rsqrt(deg)
    g = jnp.dot(x_ref[...], w1_ref[...],
                preferred_element_type=jnp.float32,
                precision=lax.Precision.HIGHEST)
    gs_ref[...] = g * dinv[:, 0:1]
    d16_ref[...] = dinv


_scale_call = pl.pallas_call(
    _scale_body,
    out_shape=(
        jax.ShapeDtypeStruct((NPAD, D), jnp.float32),
        jax.ShapeDtypeStruct((NPAD, 16), jnp.float32),
    ),
)


def _mid_body(accp_ref, gs_ref, d16_ref, b1_ref, w2_ref, wp_ref,
              u_ref, self_ref):
    acc = accp_ref[0] + accp_ref[1] + gs_ref[...]
    dinv = d16_ref[:, 0:1]
    h1 = jnp.maximum(acc * dinv + b1_ref[...], 0.0)
    w = jnp.dot(w2_ref[...], wp_ref[...],
                preferred_element_type=jnp.float32,
                precision=lax.Precision.HIGHEST)
    t = jnp.dot(h1, w, preferred_element_type=jnp.float32,
                precision=lax.Precision.HIGHEST)
    row = lax.broadcasted_iota(jnp.int32, (NPAD, 1), 0)
    u = jnp.where(row < N, t * dinv, 0.0)
    u_ref[...] = u
    self_ref[...] = jnp.sum(u * dinv, keepdims=True)


_mid_call = pl.pallas_call(
    _mid_body,
    out_shape=(
        jax.ShapeDtypeStruct((NPAD, 1), jnp.float32),
        jax.ShapeDtypeStruct((1, 1), jnp.float32),
    ),
)


def _combine_body(part_ref, self_ref, b2_ref, wp_ref, bp_ref, out_ref):
    edge_sum = jnp.sum(part_ref[...])
    const = jnp.dot(b2_ref[...], wp_ref[...],
                    preferred_element_type=jnp.float32,
                    precision=lax.Precision.HIGHEST)
    out_ref[...] = (edge_sum + self_ref[...]) / float(N) + const + bp_ref[...]


_combine_call = pl.pallas_call(
    _combine_body,
    out_shape=jax.ShapeDtypeStruct((1, 1), jnp.float32),
)


@jax.jit
def kernel(x, edge_index, W1, b1, W2, b2, Wp, bp):
    src = edge_index[0].astype(jnp.int32)
    dst = edge_index[1].astype(jnp.int32)
    pad = jnp.full((E_PAD - E,), N, dtype=jnp.int32)
    src_flat = jnp.concatenate([src, pad])
    dst_flat = jnp.concatenate([dst, pad])
    src_ch = src_flat.reshape(NC * NS, NCH, B)
    dst_ch = dst_flat.reshape(NC * NS, NCH, B)
    src_16 = src_flat.reshape(NC * NS, NCH * B)
    dst_16 = dst_flat.reshape(NC * NS, NCH * B)

    x_pad = jnp.pad(x, ((0, NPAD - N), (0, 0)))
    zflat = jnp.zeros((NPAD,), jnp.float32)
    z128 = jnp.zeros((NPAD, D), jnp.float32)

    deg_p = _deg_kernel(dst_16, zflat)
    deg16 = jnp.broadcast_to((deg_p[0] + deg_p[1])[:, None], (NPAD, 16))
    gs, d16 = _scale_call(x_pad, W1, deg16)
    acc_p = _edge_kernel(gs, src_ch, dst_ch, z128)
    u, selfterm = _mid_call(acc_p, gs, d16, b1.reshape(1, D), W2, Wp)
    part = _csum_kernel(u.reshape(NPAD), d16[:, 0], src_16, dst_16)
    out = _combine_call(part, selfterm, b2.reshape(1, D), Wp,
                        bp.reshape(1, 1))
    return out.reshape(1)
